# baseline (speedup 1.0000x reference)
import math
import functools

import jax
import jax.numpy as jnp
from jax import lax
from jax.experimental import pallas as pl
from jax.experimental.pallas import tpu as pltpu
from jax.experimental.pallas import tpu_sc as plsc

B, S, IN = 1, 2048, 32
D, H, DH = 768, 12, 64
E, DFF = 8, 3072
SCALE = 1.0 / math.sqrt(DH)

BLK = 256
NB = (2 * S) // BLK + E
P = NB * BLK


def _gelu(x):
    return 0.5 * x * (1.0 + jax.lax.erf(x * 0.7071067811865476))


def _ln(x, g, b):
    m = jnp.mean(x, axis=-1, keepdims=True)
    v = jnp.mean((x - m) ** 2, axis=-1, keepdims=True)
    return (x - m) * jax.lax.rsqrt(v + 1e-5) * g + b


def _k1a_body(x_ref, pe_ref, wproj_ref, bproj_ref, wqkv_ref, bqkv_ref,
              h_ref, q_ref, k_ref, v_ref):
    x = x_ref[...]
    h = jax.lax.dot_general(x, wproj_ref[...], (((1,), (1,)), ((), ())))
    h = h + bproj_ref[...] + pe_ref[...]
    h_ref[...] = h
    for hh in range(H // 2):
        for off, ref in ((0, q_ref), (D, k_ref), (2 * D, v_ref)):
            w = wqkv_ref[off + hh * 2 * DH:off + (hh + 1) * 2 * DH, :]
            b = bqkv_ref[:, off + hh * 2 * DH:off + (hh + 1) * 2 * DH]
            qq = jax.lax.dot_general(
                h, w, (((1,), (1,)), ((), ()))) + b
            ref[2 * hh] = qq[:, :DH]
            ref[2 * hh + 1] = qq[:, DH:]


def _k1b_body(q_ref, k_ref, v_ref, ao_ref):
    s = jax.lax.dot_general(q_ref[0], k_ref[0],
                            (((1,), (1,)), ((), ()))) * SCALE
    p = jax.nn.softmax(s, axis=-1)
    ao_ref[0] = jnp.dot(p, v_ref[0])


def _k1c_body(h_ref, ao_ref, wo_ref, bo_ref, g1_ref, be1_ref, gatew_ref,
              h1_ref, meta_ref, perm_ref, beoff_ref):
    ao = bo_ref[...]
    for hh in range(H):
        ao = ao + jax.lax.dot_general(
            ao_ref[hh], wo_ref[:, hh * DH:(hh + 1) * DH],
            (((1,), (1,)), ((), ())))
    h1 = _ln(h_ref[...] + ao, g1_ref[...], be1_ref[...])
    h1_ref[...] = h1

    logits = jax.lax.dot_general(h1, gatew_ref[...], (((1,), (1,)), ((), ())))
    probs = jax.nn.softmax(logits, axis=-1)
    lane = jax.lax.broadcasted_iota(jnp.int32, (S, E), 1)
    m1 = jnp.max(probs, axis=-1, keepdims=True)
    i1 = jnp.min(jnp.where(probs == m1, lane, E), axis=-1, keepdims=True)
    probs2 = jnp.where(lane == i1, -1.0, probs)
    m2 = jnp.max(probs2, axis=-1, keepdims=True)
    i2 = jnp.min(jnp.where(probs2 == m2, lane, E), axis=-1, keepdims=True)
    denom = m1 + m2 + 1e-8
    wA = m1 / denom
    wB = m2 / denom

    a = jnp.where((lane == i1) | (lane == i2), 1.0, 0.0)
    ltri = jnp.where(
        jax.lax.broadcasted_iota(jnp.int32, (S, S), 0)
        > jax.lax.broadcasted_iota(jnp.int32, (S, S), 1), 1.0, 0.0)
    rank = jax.lax.dot_general(ltri, a,
                               (((1,), (0,)), ((), ())))
    counts = jnp.sum(a, axis=0, keepdims=True)
    cpad = jnp.ceil(counts / BLK) * BLK
    offs = jnp.zeros_like(cpad)
    for e in range(1, E):
        offs = offs + jnp.where(lane[:1] >= e, cpad[:, e - 1:e], 0.0)
    oh1 = jnp.where(lane == i1, 1.0, 0.0)
    oh2 = jnp.where(lane == i2, 1.0, 0.0)
    d1 = jnp.sum(oh1 * (rank + offs), axis=-1, keepdims=True)
    d2 = jnp.sum(oh2 * (rank + offs), axis=-1, keepdims=True)
    lane_p = jax.lax.broadcasted_iota(jnp.int32, (S, 128), 1)
    meta = jnp.where(lane_p == 0, d1, 0.0)
    meta = jnp.where(lane_p == 1, d2, meta)
    meta = jnp.where(lane_p == 2, wA, meta)
    meta = jnp.where(lane_p == 3, wB, meta)
    meta_ref[...] = meta

    tids = jax.lax.broadcasted_iota(jnp.int32, (1, S), 1)
    thi = (tids // 128).astype(jnp.float32)
    tlo = (tids % 128).astype(jnp.float32)
    lane_b = jax.lax.broadcasted_iota(jnp.int32, (S, BLK), 1).astype(jnp.float32)
    dd = (((1,), (0,)), ((), ()))
    for pb in range(NB):
        pv = lane_b + (pb * BLK)
        ind = jnp.where((d1 == pv) | (d2 == pv), 1.0, 0.0)
        pr = (jax.lax.dot_general(thi, ind, dd) * 128.0
              + jax.lax.dot_general(tlo, ind, dd))
        perm_ref[pb:pb + 1, :] = pr.astype(jnp.int32)

    coli = jax.lax.broadcasted_iota(jnp.int32, (8, 128), 1).astype(jnp.float32) * BLK
    be = jnp.zeros((8, 128), jnp.float32)
    offs_b = offs + cpad
    for e in range(1, E):
        be = be + jnp.where(coli >= offs_b[:1, e - 1:e], 1.0, 0.0)
    nbu = jnp.sum(cpad, axis=-1, keepdims=True) / BLK
    beoff_ref[...] = jnp.where(
        jax.lax.broadcasted_iota(jnp.int32, (8, 128), 0) == 0,
        be, nbu).astype(jnp.int32)


NWORK = 32
GCHUNK = 64


def _sc_gather(table, idx):
    M = idx.shape[0]
    per_w = M // NWORK
    nch = per_w // GCHUNK
    mesh = plsc.VectorSubcoreMesh(core_axis_name="c", subcore_axis_name="s")

    @functools.partial(
        pl.kernel, mesh=mesh,
        out_type=jax.ShapeDtypeStruct((M, D), jnp.float32),
        scratch_types=[
            pltpu.VMEM((per_w,), jnp.int32),
            pltpu.VMEM((GCHUNK, D), jnp.float32),
            pltpu.VMEM((GCHUNK, D), jnp.float32),
            pltpu.SemaphoreType.DMA,
            pltpu.SemaphoreType.DMA,
        ],
    )
    def k(table_hbm, idx_hbm, out_hbm, idx_v, rows0, rows1, sem0, sem1):
        wid = lax.axis_index("s") * 2 + lax.axis_index("c")
        base = wid * per_w
        pltpu.sync_copy(idx_hbm.at[pl.ds(base, per_w)], idx_v)
        bufs = (rows0, rows1)
        sems = (sem0, sem1)
        cps = []
        for c in range(nch):
            cps.append(pltpu.async_copy(
                table_hbm.at[idx_v.at[pl.ds(c * GCHUNK, GCHUNK)]],
                bufs[c % 2], sems[c % 2]))
            if c >= 1:
                cps[c - 1].wait()
                pltpu.sync_copy(bufs[(c - 1) % 2],
                                out_hbm.at[pl.ds(base + (c - 1) * GCHUNK,
                                                 GCHUNK)])
        cps[nch - 1].wait()
        pltpu.sync_copy(bufs[(nch - 1) % 2],
                        out_hbm.at[pl.ds(base + (nch - 1) * GCHUNK, GCHUNK)])

    return k(table, idx)


def _sc_gather_pair(table, idx1, idx2):
    M = idx1.shape[0]
    per_w = M // NWORK
    mesh = plsc.VectorSubcoreMesh(core_axis_name="c", subcore_axis_name="s")

    @functools.partial(
        pl.kernel, mesh=mesh,
        out_type=[jax.ShapeDtypeStruct((M, D), jnp.float32),
                  jax.ShapeDtypeStruct((M, D), jnp.float32)],
        scratch_types=[
            pltpu.VMEM((per_w,), jnp.int32),
            pltpu.VMEM((per_w,), jnp.int32),
            pltpu.VMEM((per_w, D), jnp.float32),
            pltpu.VMEM((per_w, D), jnp.float32),
            pltpu.SemaphoreType.DMA,
            pltpu.SemaphoreType.DMA,
        ],
    )
    def k(table_hbm, i1_hbm, i2_hbm, o1_hbm, o2_hbm,
          i1_v, i2_v, r1_v, r2_v, sem1, sem2):
        wid = lax.axis_index("s") * 2 + lax.axis_index("c")
        base = wid * per_w
        pltpu.sync_copy(i1_hbm.at[pl.ds(base, per_w)], i1_v)
        pltpu.sync_copy(i2_hbm.at[pl.ds(base, per_w)], i2_v)
        cp1 = pltpu.async_copy(table_hbm.at[i1_v], r1_v, sem1)
        cp2 = pltpu.async_copy(table_hbm.at[i2_v], r2_v, sem2)
        cp1.wait()
        pltpu.sync_copy(r1_v, o1_hbm.at[pl.ds(base, per_w)])
        cp2.wait()
        pltpu.sync_copy(r2_v, o2_hbm.at[pl.ds(base, per_w)])

    return k(table, idx1, idx2)


def _k2_body(be_ref, xg_ref, w1_ref, b1_ref, w2_ref, b2_ref, out_ref):
    i = pl.program_id(0)

    @pl.when(i < be_ref[NB])
    def _():
        x = xg_ref[...].astype(jnp.bfloat16)
        w1 = w1_ref[0].astype(jnp.bfloat16)
        hid = jax.lax.dot_general(x, w1, (((1,), (1,)), ((), ())),
                                  preferred_element_type=jnp.float32)
        hid = _gelu(hid + b1_ref[0])
        w2 = w2_ref[0].astype(jnp.bfloat16)
        ye = jax.lax.dot_general(hid.astype(jnp.bfloat16), w2,
                                 (((1,), (1,)), ((), ())),
                                 preferred_element_type=jnp.float32)
        out_ref[...] = ye + b2_ref[0]


def _k3_body(h1_ref, r1_ref, r2_ref, meta_ref, g2_ref, be2_ref, gc_ref,
             bc_ref, wc1_ref, bc1_ref, wc2_ref, bc2_ref, out_ref):
    wa = meta_ref[:, 2:3]
    wb = meta_ref[:, 3:4]
    moe = wa * r1_ref[...] + wb * r2_ref[...]
    h2 = _ln(h1_ref[...] + moe, g2_ref[...], be2_ref[...])
    pooled = jnp.mean(h2, axis=0, keepdims=True)
    c = _ln(pooled, gc_ref[...], bc_ref[...])
    c = jax.lax.dot_general(c, wc1_ref[...], (((1,), (1,)), ((), ())))
    c = _gelu(c + bc1_ref[...])
    out_ref[...] = jnp.sum(c * wc2_ref[...], axis=-1, keepdims=True) + bc2_ref[...]


def _pe_table():
    position = jnp.arange(S, dtype=jnp.float32)[:, None]
    div_term = jnp.exp(jnp.arange(0, D, 2, dtype=jnp.float32)
                       * (-math.log(10000.0) / D))
    pe = jnp.zeros((S, D), dtype=jnp.float32)
    pe = pe.at[:, 0::2].set(jnp.sin(position * div_term))
    pe = pe.at[:, 1::2].set(jnp.cos(position * div_term))
    return pe


def kernel(x, Wproj, bproj, Wqkv, bqkv, Wo, bo, g1, be1, gateW, W1, b1,
           W2, b2, g2, be2, gc, bc, Wc1, bc1, Wc2, bc2):
    xr = x.reshape(S, IN)
    pe = _pe_table()
    f32 = jnp.float32
    h, q3, k3, v3 = pl.pallas_call(
        _k1a_body,
        out_shape=[jax.ShapeDtypeStruct((S, D), f32),
                   jax.ShapeDtypeStruct((H, S, DH), f32),
                   jax.ShapeDtypeStruct((H, S, DH), f32),
                   jax.ShapeDtypeStruct((H, S, DH), f32)],
    )(xr, pe, Wproj, bproj.reshape(1, D), Wqkv, bqkv.reshape(1, 3 * D))

    ao = pl.pallas_call(
        _k1b_body,
        grid=(H,),
        in_specs=[
            pl.BlockSpec((1, S, DH), lambda hh: (hh, 0, 0)),
            pl.BlockSpec((1, S, DH), lambda hh: (hh, 0, 0)),
            pl.BlockSpec((1, S, DH), lambda hh: (hh, 0, 0)),
        ],
        out_specs=pl.BlockSpec((1, S, DH), lambda hh: (hh, 0, 0)),
        out_shape=jax.ShapeDtypeStruct((H, S, DH), f32),
        compiler_params=pltpu.CompilerParams(
            dimension_semantics=("parallel",)),
    )(q3, k3, v3)

    h1, meta, perm, beoff = pl.pallas_call(
        _k1c_body,
        out_shape=[jax.ShapeDtypeStruct((S, D), f32),
                   jax.ShapeDtypeStruct((S, 128), f32),
                   jax.ShapeDtypeStruct((NB, BLK), jnp.int32),
                   jax.ShapeDtypeStruct((8, 128), jnp.int32)],
    )(h, ao, Wo, bo.reshape(1, D), g1.reshape(1, D), be1.reshape(1, D),
      gateW)

    be_nbu = jnp.concatenate(
        [beoff[0, :NB], beoff[1, :1]]).astype(jnp.int32)
    perm_flat = perm.reshape(P)
    d1 = meta[:, 0].astype(jnp.int32)
    d2 = meta[:, 1].astype(jnp.int32)

    xg = _sc_gather(h1, perm_flat)

    moe_s = pl.pallas_call(
        _k2_body,
        grid_spec=pltpu.PrefetchScalarGridSpec(
            num_scalar_prefetch=1,
            grid=(NB,),
            in_specs=[
                pl.BlockSpec((BLK, D), lambda i, be: (i, 0)),
                pl.BlockSpec((1, DFF, D), lambda i, be: (be[i], 0, 0)),
                pl.BlockSpec((1, 1, DFF), lambda i, be: (be[i], 0, 0)),
                pl.BlockSpec((1, D, DFF), lambda i, be: (be[i], 0, 0)),
                pl.BlockSpec((1, 1, D), lambda i, be: (be[i], 0, 0)),
            ],
            out_specs=pl.BlockSpec((BLK, D), lambda i, be: (i, 0)),
        ),
        out_shape=jax.ShapeDtypeStruct((P, D), f32),
        compiler_params=pltpu.CompilerParams(
            dimension_semantics=("arbitrary",)),
    )(be_nbu, xg, W1, b1.reshape(E, 1, DFF), W2, b2.reshape(E, 1, D))

    r1, r2 = _sc_gather_pair(moe_s, d1, d2)

    out = pl.pallas_call(
        _k3_body,
        out_shape=jax.ShapeDtypeStruct((1, 1), f32),
    )(h1, r1, r2, meta, g2.reshape(1, D), be2.reshape(1, D),
      gc.reshape(1, D), bc.reshape(1, D), Wc1, bc1.reshape(1, D // 2),
      Wc2, bc2.reshape(1, 1))
    return out.reshape(B)


# speedup vs baseline: 1.0044x; 1.0044x over previous
import math
import functools

import jax
import jax.numpy as jnp
from jax import lax
from jax.experimental import pallas as pl
from jax.experimental.pallas import tpu as pltpu
from jax.experimental.pallas import tpu_sc as plsc

B, S, IN = 1, 2048, 32
D, H, DH = 768, 12, 64
E, DFF = 8, 3072
SCALE = 1.0 / math.sqrt(DH)

BLK = 256
NB = (2 * S) // BLK + E
P = NB * BLK


def _gelu(x):
    return 0.5 * x * (1.0 + jax.lax.erf(x * 0.7071067811865476))


def _ln(x, g, b):
    m = jnp.mean(x, axis=-1, keepdims=True)
    v = jnp.mean((x - m) ** 2, axis=-1, keepdims=True)
    return (x - m) * jax.lax.rsqrt(v + 1e-5) * g + b


def _k1a_body(x_ref, pe_ref, wproj_ref, bproj_ref, wqkv_ref, bqkv_ref,
              h_ref, q_ref, k_ref, v_ref):
    x = x_ref[...]
    h = jax.lax.dot_general(x, wproj_ref[...], (((1,), (1,)), ((), ())))
    h = h + bproj_ref[...] + pe_ref[...]
    h_ref[...] = h
    for hh in range(H // 2):
        for off, ref in ((0, q_ref), (D, k_ref), (2 * D, v_ref)):
            w = wqkv_ref[off + hh * 2 * DH:off + (hh + 1) * 2 * DH, :]
            b = bqkv_ref[:, off + hh * 2 * DH:off + (hh + 1) * 2 * DH]
            qq = jax.lax.dot_general(
                h, w, (((1,), (1,)), ((), ()))) + b
            ref[2 * hh] = qq[:, :DH]
            ref[2 * hh + 1] = qq[:, DH:]


def _k1b_body(q_ref, k_ref, v_ref, ao_ref):
    s = jax.lax.dot_general(q_ref[0], k_ref[0],
                            (((1,), (1,)), ((), ()))) * SCALE
    p = jax.nn.softmax(s, axis=-1)
    ao_ref[0] = jnp.dot(p, v_ref[0])


def _k1c_body(h_ref, ao_ref, wo_ref, bo_ref, g1_ref, be1_ref, gatew_ref,
              h1_ref, meta_ref, perm_ref, beoff_ref):
    ao = bo_ref[...]
    for hh in range(H):
        ao = ao + jax.lax.dot_general(
            ao_ref[hh], wo_ref[:, hh * DH:(hh + 1) * DH],
            (((1,), (1,)), ((), ())))
    h1 = _ln(h_ref[...] + ao, g1_ref[...], be1_ref[...])
    h1_ref[...] = h1

    logits = jax.lax.dot_general(h1, gatew_ref[...], (((1,), (1,)), ((), ())))
    probs = jax.nn.softmax(logits, axis=-1)
    lane = jax.lax.broadcasted_iota(jnp.int32, (S, E), 1)
    m1 = jnp.max(probs, axis=-1, keepdims=True)
    i1 = jnp.min(jnp.where(probs == m1, lane, E), axis=-1, keepdims=True)
    probs2 = jnp.where(lane == i1, -1.0, probs)
    m2 = jnp.max(probs2, axis=-1, keepdims=True)
    i2 = jnp.min(jnp.where(probs2 == m2, lane, E), axis=-1, keepdims=True)
    denom = m1 + m2 + 1e-8
    wA = m1 / denom
    wB = m2 / denom

    a = jnp.where((lane == i1) | (lane == i2), 1.0, 0.0)
    ltri = jnp.where(
        jax.lax.broadcasted_iota(jnp.int32, (S, S), 0)
        > jax.lax.broadcasted_iota(jnp.int32, (S, S), 1), 1.0, 0.0)
    rank = jax.lax.dot_general(ltri, a,
                               (((1,), (0,)), ((), ())))
    counts = jnp.sum(a, axis=0, keepdims=True)
    cpad = jnp.ceil(counts / BLK) * BLK
    offs = jnp.zeros_like(cpad)
    for e in range(1, E):
        offs = offs + jnp.where(lane[:1] >= e, cpad[:, e - 1:e], 0.0)
    oh1 = jnp.where(lane == i1, 1.0, 0.0)
    oh2 = jnp.where(lane == i2, 1.0, 0.0)
    d1 = jnp.sum(oh1 * (rank + offs), axis=-1, keepdims=True)
    d2 = jnp.sum(oh2 * (rank + offs), axis=-1, keepdims=True)
    lane_p = jax.lax.broadcasted_iota(jnp.int32, (S, 128), 1)
    meta = jnp.where(lane_p == 0, d1, 0.0)
    meta = jnp.where(lane_p == 1, d2, meta)
    meta = jnp.where(lane_p == 2, wA, meta)
    meta = jnp.where(lane_p == 3, wB, meta)
    meta_ref[...] = meta

    tids = jax.lax.broadcasted_iota(jnp.int32, (1, S), 1)
    thi = (tids // 128).astype(jnp.float32)
    tlo = (tids % 128).astype(jnp.float32)
    lane_b = jax.lax.broadcasted_iota(jnp.int32, (S, BLK), 1).astype(jnp.float32)
    dd = (((1,), (0,)), ((), ()))
    for pb in range(NB):
        pv = lane_b + (pb * BLK)
        ind = jnp.where((d1 == pv) | (d2 == pv), 1.0, 0.0)
        pr = (jax.lax.dot_general(thi, ind, dd) * 128.0
              + jax.lax.dot_general(tlo, ind, dd))
        perm_ref[pb:pb + 1, :] = pr.astype(jnp.int32)

    coli = jax.lax.broadcasted_iota(jnp.int32, (8, 128), 1).astype(jnp.float32) * BLK
    be = jnp.zeros((8, 128), jnp.float32)
    offs_b = offs + cpad
    for e in range(1, E):
        be = be + jnp.where(coli >= offs_b[:1, e - 1:e], 1.0, 0.0)
    nbu = jnp.sum(cpad, axis=-1, keepdims=True) / BLK
    beoff_ref[...] = jnp.where(
        jax.lax.broadcasted_iota(jnp.int32, (8, 128), 0) == 0,
        be, nbu).astype(jnp.int32)


NWORK = 32
GCHUNK = 64


def _sc_gather(table, idx):
    M = idx.shape[0]
    per_w = M // NWORK
    nch = per_w // GCHUNK
    mesh = plsc.VectorSubcoreMesh(core_axis_name="c", subcore_axis_name="s")

    @functools.partial(
        pl.kernel, mesh=mesh,
        out_type=jax.ShapeDtypeStruct((M, D), jnp.float32),
        scratch_types=[
            pltpu.VMEM((per_w,), jnp.int32),
            pltpu.VMEM((GCHUNK, D), jnp.float32),
            pltpu.VMEM((GCHUNK, D), jnp.float32),
            pltpu.SemaphoreType.DMA,
            pltpu.SemaphoreType.DMA,
        ],
    )
    def k(table_hbm, idx_hbm, out_hbm, idx_v, rows0, rows1, sem0, sem1):
        wid = lax.axis_index("s") * 2 + lax.axis_index("c")
        base = wid * per_w
        pltpu.sync_copy(idx_hbm.at[pl.ds(base, per_w)], idx_v)
        bufs = (rows0, rows1)
        sems = (sem0, sem1)
        cps = []
        for c in range(nch):
            cps.append(pltpu.async_copy(
                table_hbm.at[idx_v.at[pl.ds(c * GCHUNK, GCHUNK)]],
                bufs[c % 2], sems[c % 2]))
            if c >= 1:
                cps[c - 1].wait()
                pltpu.sync_copy(bufs[(c - 1) % 2],
                                out_hbm.at[pl.ds(base + (c - 1) * GCHUNK,
                                                 GCHUNK)])
        cps[nch - 1].wait()
        pltpu.sync_copy(bufs[(nch - 1) % 2],
                        out_hbm.at[pl.ds(base + (nch - 1) * GCHUNK, GCHUNK)])

    return k(table, idx)


def _sc_gather_pair(table, idx1, idx2):
    M = idx1.shape[0]
    per_w = M // NWORK
    mesh = plsc.VectorSubcoreMesh(core_axis_name="c", subcore_axis_name="s")

    @functools.partial(
        pl.kernel, mesh=mesh,
        out_type=[jax.ShapeDtypeStruct((M, D), jnp.float32),
                  jax.ShapeDtypeStruct((M, D), jnp.float32)],
        scratch_types=[
            pltpu.VMEM((per_w,), jnp.int32),
            pltpu.VMEM((per_w,), jnp.int32),
            pltpu.VMEM((per_w, D), jnp.float32),
            pltpu.VMEM((per_w, D), jnp.float32),
            pltpu.SemaphoreType.DMA,
            pltpu.SemaphoreType.DMA,
        ],
    )
    def k(table_hbm, i1_hbm, i2_hbm, o1_hbm, o2_hbm,
          i1_v, i2_v, r1_v, r2_v, sem1, sem2):
        wid = lax.axis_index("s") * 2 + lax.axis_index("c")
        base = wid * per_w
        pltpu.sync_copy(i1_hbm.at[pl.ds(base, per_w)], i1_v)
        pltpu.sync_copy(i2_hbm.at[pl.ds(base, per_w)], i2_v)
        cp1 = pltpu.async_copy(table_hbm.at[i1_v], r1_v, sem1)
        cp2 = pltpu.async_copy(table_hbm.at[i2_v], r2_v, sem2)
        cp1.wait()
        pltpu.sync_copy(r1_v, o1_hbm.at[pl.ds(base, per_w)])
        cp2.wait()
        pltpu.sync_copy(r2_v, o2_hbm.at[pl.ds(base, per_w)])

    return k(table, idx1, idx2)


def _k2_body(be_ref, xg_ref, w1_ref, b1_ref, w2_ref, b2_ref, out_ref):
    i = pl.program_id(0)

    @pl.when(i < be_ref[NB])
    def _():
        x = xg_ref[...]
        hid = jax.lax.dot_general(x, w1_ref[0], (((1,), (1,)), ((), ())))
        hid = _gelu(hid + b1_ref[0])
        ye = jax.lax.dot_general(hid, w2_ref[0], (((1,), (1,)), ((), ())))
        out_ref[...] = ye + b2_ref[0]


def _k3_body(h1_ref, r1_ref, r2_ref, meta_ref, g2_ref, be2_ref, gc_ref,
             bc_ref, wc1_ref, bc1_ref, wc2_ref, bc2_ref, out_ref):
    wa = meta_ref[:, 2:3]
    wb = meta_ref[:, 3:4]
    moe = wa * r1_ref[...] + wb * r2_ref[...]
    h2 = _ln(h1_ref[...] + moe, g2_ref[...], be2_ref[...])
    pooled = jnp.mean(h2, axis=0, keepdims=True)
    c = _ln(pooled, gc_ref[...], bc_ref[...])
    c = jax.lax.dot_general(c, wc1_ref[...], (((1,), (1,)), ((), ())))
    c = _gelu(c + bc1_ref[...])
    out_ref[...] = jnp.sum(c * wc2_ref[...], axis=-1, keepdims=True) + bc2_ref[...]


def _pe_table():
    position = jnp.arange(S, dtype=jnp.float32)[:, None]
    div_term = jnp.exp(jnp.arange(0, D, 2, dtype=jnp.float32)
                       * (-math.log(10000.0) / D))
    pe = jnp.zeros((S, D), dtype=jnp.float32)
    pe = pe.at[:, 0::2].set(jnp.sin(position * div_term))
    pe = pe.at[:, 1::2].set(jnp.cos(position * div_term))
    return pe


def kernel(x, Wproj, bproj, Wqkv, bqkv, Wo, bo, g1, be1, gateW, W1, b1,
           W2, b2, g2, be2, gc, bc, Wc1, bc1, Wc2, bc2):
    xr = x.reshape(S, IN)
    pe = _pe_table()
    f32 = jnp.float32
    h, q3, k3, v3 = pl.pallas_call(
        _k1a_body,
        out_shape=[jax.ShapeDtypeStruct((S, D), f32),
                   jax.ShapeDtypeStruct((H, S, DH), f32),
                   jax.ShapeDtypeStruct((H, S, DH), f32),
                   jax.ShapeDtypeStruct((H, S, DH), f32)],
    )(xr, pe, Wproj, bproj.reshape(1, D), Wqkv, bqkv.reshape(1, 3 * D))

    ao = pl.pallas_call(
        _k1b_body,
        grid=(H,),
        in_specs=[
            pl.BlockSpec((1, S, DH), lambda hh: (hh, 0, 0)),
            pl.BlockSpec((1, S, DH), lambda hh: (hh, 0, 0)),
            pl.BlockSpec((1, S, DH), lambda hh: (hh, 0, 0)),
        ],
        out_specs=pl.BlockSpec((1, S, DH), lambda hh: (hh, 0, 0)),
        out_shape=jax.ShapeDtypeStruct((H, S, DH), f32),
        compiler_params=pltpu.CompilerParams(
            dimension_semantics=("parallel",)),
    )(q3, k3, v3)

    h1, meta, perm, beoff = pl.pallas_call(
        _k1c_body,
        out_shape=[jax.ShapeDtypeStruct((S, D), f32),
                   jax.ShapeDtypeStruct((S, 128), f32),
                   jax.ShapeDtypeStruct((NB, BLK), jnp.int32),
                   jax.ShapeDtypeStruct((8, 128), jnp.int32)],
    )(h, ao, Wo, bo.reshape(1, D), g1.reshape(1, D), be1.reshape(1, D),
      gateW)

    be_nbu = jnp.concatenate(
        [beoff[0, :NB], beoff[1, :1]]).astype(jnp.int32)
    perm_flat = perm.reshape(P)
    d1 = meta[:, 0].astype(jnp.int32)
    d2 = meta[:, 1].astype(jnp.int32)

    xg = _sc_gather(h1, perm_flat)

    moe_s = pl.pallas_call(
        _k2_body,
        grid_spec=pltpu.PrefetchScalarGridSpec(
            num_scalar_prefetch=1,
            grid=(NB,),
            in_specs=[
                pl.BlockSpec((BLK, D), lambda i, be: (i, 0)),
                pl.BlockSpec((1, DFF, D), lambda i, be: (be[i], 0, 0)),
                pl.BlockSpec((1, 1, DFF), lambda i, be: (be[i], 0, 0)),
                pl.BlockSpec((1, D, DFF), lambda i, be: (be[i], 0, 0)),
                pl.BlockSpec((1, 1, D), lambda i, be: (be[i], 0, 0)),
            ],
            out_specs=pl.BlockSpec((BLK, D), lambda i, be: (i, 0)),
        ),
        out_shape=jax.ShapeDtypeStruct((P, D), f32),
        compiler_params=pltpu.CompilerParams(
            dimension_semantics=("arbitrary",)),
    )(be_nbu, xg, W1, b1.reshape(E, 1, DFF), W2, b2.reshape(E, 1, D))

    r1, r2 = _sc_gather_pair(moe_s, d1, d2)

    out = pl.pallas_call(
        _k3_body,
        out_shape=jax.ShapeDtypeStruct((1, 1), f32),
    )(h1, r1, r2, meta, g2.reshape(1, D), be2.reshape(1, D),
      gc.reshape(1, D), bc.reshape(1, D), Wc1, bc1.reshape(1, D // 2),
      Wc2, bc2.reshape(1, 1))
    return out.reshape(B)


# speedup vs baseline: 1.0915x; 1.0867x over previous
import math
import functools

import jax
import jax.numpy as jnp
from jax import lax
from jax.experimental import pallas as pl
from jax.experimental.pallas import tpu as pltpu
from jax.experimental.pallas import tpu_sc as plsc

B, S, IN = 1, 2048, 32
D, H, DH = 768, 12, 64
E, DFF = 8, 3072
SCALE = 1.0 / math.sqrt(DH)

BLK = 256
NB = (2 * S) // BLK + E
P = NB * BLK


def _gelu(x):
    return 0.5 * x * (1.0 + jax.lax.erf(x * 0.7071067811865476))


def _ln(x, g, b):
    m = jnp.mean(x, axis=-1, keepdims=True)
    v = jnp.mean((x - m) ** 2, axis=-1, keepdims=True)
    return (x - m) * jax.lax.rsqrt(v + 1e-5) * g + b


def _k1a_body(x_ref, pe_ref, wproj_ref, bproj_ref, wqkv_ref, bqkv_ref,
              h_ref, q_ref, k_ref, v_ref):
    x = x_ref[...]
    h = jax.lax.dot_general(x, wproj_ref[...], (((1,), (1,)), ((), ())))
    h = h + bproj_ref[...] + pe_ref[...]
    h_ref[...] = h
    for hh in range(H // 2):
        for off, ref in ((0, q_ref), (D, k_ref), (2 * D, v_ref)):
            w = wqkv_ref[off + hh * 2 * DH:off + (hh + 1) * 2 * DH, :]
            b = bqkv_ref[:, off + hh * 2 * DH:off + (hh + 1) * 2 * DH]
            qq = jax.lax.dot_general(
                h, w, (((1,), (1,)), ((), ()))) + b
            ref[2 * hh] = qq[:, :DH]
            ref[2 * hh + 1] = qq[:, DH:]


def _k1b_body(q_ref, k_ref, v_ref, ao_ref):
    s = jax.lax.dot_general(q_ref[0], k_ref[0],
                            (((1,), (1,)), ((), ()))) * SCALE
    p = jax.nn.softmax(s, axis=-1)
    ao_ref[0] = jnp.dot(p, v_ref[0])


def _k1c_body(h_ref, ao_ref, wo_ref, bo_ref, g1_ref, be1_ref, gatew_ref,
              h1_ref, meta_ref, perm_ref, beoff_ref):
    ao = bo_ref[...]
    for hh in range(H):
        ao = ao + jax.lax.dot_general(
            ao_ref[hh], wo_ref[:, hh * DH:(hh + 1) * DH],
            (((1,), (1,)), ((), ())))
    h1 = _ln(h_ref[...] + ao, g1_ref[...], be1_ref[...])
    h1_ref[...] = h1

    logits = jax.lax.dot_general(h1, gatew_ref[...], (((1,), (1,)), ((), ())))
    probs = jax.nn.softmax(logits, axis=-1)
    lane = jax.lax.broadcasted_iota(jnp.int32, (S, E), 1)
    m1 = jnp.max(probs, axis=-1, keepdims=True)
    i1 = jnp.min(jnp.where(probs == m1, lane, E), axis=-1, keepdims=True)
    probs2 = jnp.where(lane == i1, -1.0, probs)
    m2 = jnp.max(probs2, axis=-1, keepdims=True)
    i2 = jnp.min(jnp.where(probs2 == m2, lane, E), axis=-1, keepdims=True)
    denom = m1 + m2 + 1e-8
    wA = m1 / denom
    wB = m2 / denom

    a = jnp.where((lane == i1) | (lane == i2), 1.0, 0.0)
    ltri = jnp.where(
        jax.lax.broadcasted_iota(jnp.int32, (S, S), 0)
        > jax.lax.broadcasted_iota(jnp.int32, (S, S), 1), 1.0, 0.0)
    rank = jax.lax.dot_general(ltri, a,
                               (((1,), (0,)), ((), ())))
    counts = jnp.sum(a, axis=0, keepdims=True)
    cpad = jnp.ceil(counts / BLK) * BLK
    offs = jnp.zeros_like(cpad)
    for e in range(1, E):
        offs = offs + jnp.where(lane[:1] >= e, cpad[:, e - 1:e], 0.0)
    oh1 = jnp.where(lane == i1, 1.0, 0.0)
    oh2 = jnp.where(lane == i2, 1.0, 0.0)
    d1 = jnp.sum(oh1 * (rank + offs), axis=-1, keepdims=True)
    d2 = jnp.sum(oh2 * (rank + offs), axis=-1, keepdims=True)
    lane_p = jax.lax.broadcasted_iota(jnp.int32, (S, 128), 1)
    meta = jnp.where(lane_p == 0, d1, 0.0)
    meta = jnp.where(lane_p == 1, d2, meta)
    meta = jnp.where(lane_p == 2, wA, meta)
    meta = jnp.where(lane_p == 3, wB, meta)
    meta_ref[...] = meta

    tids = jax.lax.broadcasted_iota(jnp.int32, (1, S), 1)
    thi = (tids // 128).astype(jnp.float32)
    tlo = (tids % 128).astype(jnp.float32)
    lane_b = jax.lax.broadcasted_iota(jnp.int32, (S, BLK), 1).astype(jnp.float32)
    dd = (((1,), (0,)), ((), ()))
    for pb in range(NB):
        pv = lane_b + (pb * BLK)
        ind = jnp.where((d1 == pv) | (d2 == pv), 1.0, 0.0)
        pr = (jax.lax.dot_general(thi, ind, dd) * 128.0
              + jax.lax.dot_general(tlo, ind, dd))
        perm_ref[pb:pb + 1, :] = pr.astype(jnp.int32)

    coli = jax.lax.broadcasted_iota(jnp.int32, (8, 128), 1).astype(jnp.float32) * BLK
    be = jnp.zeros((8, 128), jnp.float32)
    offs_b = offs + cpad
    for e in range(1, E):
        be = be + jnp.where(coli >= offs_b[:1, e - 1:e], 1.0, 0.0)
    nbu = jnp.sum(cpad, axis=-1, keepdims=True) / BLK
    beoff_ref[...] = jnp.where(
        jax.lax.broadcasted_iota(jnp.int32, (8, 128), 0) == 0,
        be, nbu).astype(jnp.int32)


NWORK = 32
GCHUNK = 64


def _sc_gather(table, idx):
    M = idx.shape[0]
    per_w = M // NWORK
    nch = per_w // GCHUNK
    mesh = plsc.VectorSubcoreMesh(core_axis_name="c", subcore_axis_name="s")

    @functools.partial(
        pl.kernel, mesh=mesh,
        out_type=jax.ShapeDtypeStruct((M, D), jnp.float32),
        scratch_types=[
            pltpu.VMEM((per_w,), jnp.int32),
            pltpu.VMEM((GCHUNK, D), jnp.float32),
            pltpu.VMEM((GCHUNK, D), jnp.float32),
            pltpu.SemaphoreType.DMA,
            pltpu.SemaphoreType.DMA,
        ],
    )
    def k(table_hbm, idx_hbm, out_hbm, idx_v, rows0, rows1, sem0, sem1):
        wid = lax.axis_index("s") * 2 + lax.axis_index("c")
        base = wid * per_w
        pltpu.sync_copy(idx_hbm.at[pl.ds(base, per_w)], idx_v)
        bufs = (rows0, rows1)
        sems = (sem0, sem1)
        cps = []
        for c in range(nch):
            cps.append(pltpu.async_copy(
                table_hbm.at[idx_v.at[pl.ds(c * GCHUNK, GCHUNK)]],
                bufs[c % 2], sems[c % 2]))
            if c >= 1:
                cps[c - 1].wait()
                pltpu.sync_copy(bufs[(c - 1) % 2],
                                out_hbm.at[pl.ds(base + (c - 1) * GCHUNK,
                                                 GCHUNK)])
        cps[nch - 1].wait()
        pltpu.sync_copy(bufs[(nch - 1) % 2],
                        out_hbm.at[pl.ds(base + (nch - 1) * GCHUNK, GCHUNK)])

    return k(table, idx)


def _sc_gather_pair(table, idx1, idx2):
    M = idx1.shape[0]
    per_w = M // NWORK
    mesh = plsc.VectorSubcoreMesh(core_axis_name="c", subcore_axis_name="s")

    @functools.partial(
        pl.kernel, mesh=mesh,
        out_type=[jax.ShapeDtypeStruct((M, D), jnp.float32),
                  jax.ShapeDtypeStruct((M, D), jnp.float32)],
        scratch_types=[
            pltpu.VMEM((per_w,), jnp.int32),
            pltpu.VMEM((per_w,), jnp.int32),
            pltpu.VMEM((per_w, D), jnp.float32),
            pltpu.VMEM((per_w, D), jnp.float32),
            pltpu.SemaphoreType.DMA,
            pltpu.SemaphoreType.DMA,
        ],
    )
    def k(table_hbm, i1_hbm, i2_hbm, o1_hbm, o2_hbm,
          i1_v, i2_v, r1_v, r2_v, sem1, sem2):
        wid = lax.axis_index("s") * 2 + lax.axis_index("c")
        base = wid * per_w
        pltpu.sync_copy(i1_hbm.at[pl.ds(base, per_w)], i1_v)
        pltpu.sync_copy(i2_hbm.at[pl.ds(base, per_w)], i2_v)
        cp1 = pltpu.async_copy(table_hbm.at[i1_v], r1_v, sem1)
        cp2 = pltpu.async_copy(table_hbm.at[i2_v], r2_v, sem2)
        cp1.wait()
        pltpu.sync_copy(r1_v, o1_hbm.at[pl.ds(base, per_w)])
        cp2.wait()
        pltpu.sync_copy(r2_v, o2_hbm.at[pl.ds(base, per_w)])

    return k(table, idx1, idx2)


def _sc_gather3(table, idx3):
    M = idx3.shape[1]
    per_w = M // NWORK
    mesh = plsc.VectorSubcoreMesh(core_axis_name="c", subcore_axis_name="s")
    out_sd = jax.ShapeDtypeStruct((M, D), jnp.float32)

    @functools.partial(
        pl.kernel, mesh=mesh,
        out_type=[out_sd, out_sd, out_sd],
        scratch_types=[
            pltpu.VMEM((per_w,), jnp.int32),
            pltpu.VMEM((per_w,), jnp.int32),
            pltpu.VMEM((per_w, D), jnp.float32),
            pltpu.VMEM((per_w, D), jnp.float32),
            pltpu.SemaphoreType.DMA,
            pltpu.SemaphoreType.DMA,
        ],
    )
    def k(table_hbm, idx_hbm, o0_hbm, o1_hbm, o2_hbm,
          ia_v, ib_v, ra_v, rb_v, sema, semb):
        wid = lax.axis_index("s") * 2 + lax.axis_index("c")
        base = wid * per_w
        outs = (o0_hbm, o1_hbm, o2_hbm)
        ibufs = (ia_v, ib_v)
        rbufs = (ra_v, rb_v)
        sems = (sema, semb)
        cps = [None, None, None]
        for j in range(3):
            pltpu.sync_copy(idx_hbm.at[j, pl.ds(base, per_w)], ibufs[j % 2])
            cps[j] = pltpu.async_copy(table_hbm.at[ibufs[j % 2]],
                                      rbufs[j % 2], sems[j % 2])
            if j >= 1:
                cps[j - 1].wait()
                pltpu.sync_copy(rbufs[(j - 1) % 2],
                                outs[j - 1].at[pl.ds(base, per_w)])
        cps[2].wait()
        pltpu.sync_copy(rbufs[0], outs[2].at[pl.ds(base, per_w)])

    return k(table, idx3)


def _k2_body(be_ref, xg_ref, w1_ref, b1_ref, w2_ref, b2_ref, out_ref):
    i = pl.program_id(0)

    @pl.when(i < be_ref[NB])
    def _():
        x = xg_ref[...]
        hid = jax.lax.dot_general(x, w1_ref[0], (((1,), (1,)), ((), ())))
        hid = _gelu(hid + b1_ref[0])
        ye = jax.lax.dot_general(hid, w2_ref[0], (((1,), (1,)), ((), ())))
        out_ref[...] = ye + b2_ref[0]


def _k3_body(h1_ref, r1_ref, r2_ref, meta_ref, g2_ref, be2_ref, gc_ref,
             bc_ref, wc1_ref, bc1_ref, wc2_ref, bc2_ref, out_ref):
    wa = meta_ref[:, 2:3]
    wb = meta_ref[:, 3:4]
    moe = wa * r1_ref[...] + wb * r2_ref[...]
    h2 = _ln(h1_ref[...] + moe, g2_ref[...], be2_ref[...])
    pooled = jnp.mean(h2, axis=0, keepdims=True)
    c = _ln(pooled, gc_ref[...], bc_ref[...])
    c = jax.lax.dot_general(c, wc1_ref[...], (((1,), (1,)), ((), ())))
    c = _gelu(c + bc1_ref[...])
    out_ref[...] = jnp.sum(c * wc2_ref[...], axis=-1, keepdims=True) + bc2_ref[...]


def _pe_table():
    position = jnp.arange(S, dtype=jnp.float32)[:, None]
    div_term = jnp.exp(jnp.arange(0, D, 2, dtype=jnp.float32)
                       * (-math.log(10000.0) / D))
    pe = jnp.zeros((S, D), dtype=jnp.float32)
    pe = pe.at[:, 0::2].set(jnp.sin(position * div_term))
    pe = pe.at[:, 1::2].set(jnp.cos(position * div_term))
    return pe


def kernel(x, Wproj, bproj, Wqkv, bqkv, Wo, bo, g1, be1, gateW, W1, b1,
           W2, b2, g2, be2, gc, bc, Wc1, bc1, Wc2, bc2):
    xr = x.reshape(S, IN)
    pe = _pe_table()
    f32 = jnp.float32
    h, q3, k3, v3 = pl.pallas_call(
        _k1a_body,
        out_shape=[jax.ShapeDtypeStruct((S, D), f32),
                   jax.ShapeDtypeStruct((H, S, DH), f32),
                   jax.ShapeDtypeStruct((H, S, DH), f32),
                   jax.ShapeDtypeStruct((H, S, DH), f32)],
    )(xr, pe, Wproj, bproj.reshape(1, D), Wqkv, bqkv.reshape(1, 3 * D))

    ao = pl.pallas_call(
        _k1b_body,
        grid=(H,),
        in_specs=[
            pl.BlockSpec((1, S, DH), lambda hh: (hh, 0, 0)),
            pl.BlockSpec((1, S, DH), lambda hh: (hh, 0, 0)),
            pl.BlockSpec((1, S, DH), lambda hh: (hh, 0, 0)),
        ],
        out_specs=pl.BlockSpec((1, S, DH), lambda hh: (hh, 0, 0)),
        out_shape=jax.ShapeDtypeStruct((H, S, DH), f32),
        compiler_params=pltpu.CompilerParams(
            dimension_semantics=("parallel",)),
    )(q3, k3, v3)

    h1, meta, perm, beoff = pl.pallas_call(
        _k1c_body,
        out_shape=[jax.ShapeDtypeStruct((S, D), f32),
                   jax.ShapeDtypeStruct((S, 128), f32),
                   jax.ShapeDtypeStruct((NB, BLK), jnp.int32),
                   jax.ShapeDtypeStruct((8, 128), jnp.int32)],
    )(h, ao, Wo, bo.reshape(1, D), g1.reshape(1, D), be1.reshape(1, D),
      gateW)

    be_nbu = jnp.concatenate(
        [beoff[0, :NB], beoff[1, :1]]).astype(jnp.int32)
    perm_flat = perm.reshape(P)
    d1 = meta[:, 0].astype(jnp.int32)
    d2 = meta[:, 1].astype(jnp.int32)

    xg_parts = _sc_gather3(h1, perm.reshape(3, P // 3))
    xg = jnp.concatenate(xg_parts, axis=0)

    moe_s = pl.pallas_call(
        _k2_body,
        grid_spec=pltpu.PrefetchScalarGridSpec(
            num_scalar_prefetch=1,
            grid=(NB,),
            in_specs=[
                pl.BlockSpec((BLK, D), lambda i, be: (i, 0)),
                pl.BlockSpec((1, DFF, D), lambda i, be: (be[i], 0, 0)),
                pl.BlockSpec((1, 1, DFF), lambda i, be: (be[i], 0, 0)),
                pl.BlockSpec((1, D, DFF), lambda i, be: (be[i], 0, 0)),
                pl.BlockSpec((1, 1, D), lambda i, be: (be[i], 0, 0)),
            ],
            out_specs=pl.BlockSpec((BLK, D), lambda i, be: (i, 0)),
        ),
        out_shape=jax.ShapeDtypeStruct((P, D), f32),
        compiler_params=pltpu.CompilerParams(
            dimension_semantics=("arbitrary",)),
    )(be_nbu, xg, W1, b1.reshape(E, 1, DFF), W2, b2.reshape(E, 1, D))

    r1, r2 = _sc_gather_pair(moe_s, d1, d2)

    out = pl.pallas_call(
        _k3_body,
        out_shape=jax.ShapeDtypeStruct((1, 1), f32),
    )(h1, r1, r2, meta, g2.reshape(1, D), be2.reshape(1, D),
      gc.reshape(1, D), bc.reshape(1, D), Wc1, bc1.reshape(1, D // 2),
      Wc2, bc2.reshape(1, 1))
    return out.reshape(B)


# speedup vs baseline: 1.3454x; 1.2327x over previous
import math
import functools

import jax
import jax.numpy as jnp
from jax import lax
from jax.experimental import pallas as pl
from jax.experimental.pallas import tpu as pltpu
from jax.experimental.pallas import tpu_sc as plsc

B, S, IN = 1, 2048, 32
D, H, DH = 768, 12, 64
E, DFF = 8, 3072
SCALE = 1.0 / math.sqrt(DH)

BLK = 256
NB = (2 * S) // BLK + E
P = NB * BLK


def _gelu(x):
    return 0.5 * x * (1.0 + jax.lax.erf(x * 0.7071067811865476))


def _ln(x, g, b):
    m = jnp.mean(x, axis=-1, keepdims=True)
    v = jnp.mean((x - m) ** 2, axis=-1, keepdims=True)
    return (x - m) * jax.lax.rsqrt(v + 1e-5) * g + b


def _k1a_body(x_ref, pe_ref, wproj_ref, bproj_ref, wqkv_ref, bqkv_ref,
              h_ref, q_ref, k_ref, v_ref):
    x = x_ref[...]
    h = jax.lax.dot_general(x, wproj_ref[...], (((1,), (1,)), ((), ())))
    h = h + bproj_ref[...] + pe_ref[...]
    h_ref[...] = h
    for hh in range(H // 2):
        for off, ref in ((0, q_ref), (D, k_ref), (2 * D, v_ref)):
            w = wqkv_ref[off + hh * 2 * DH:off + (hh + 1) * 2 * DH, :]
            b = bqkv_ref[:, off + hh * 2 * DH:off + (hh + 1) * 2 * DH]
            qq = jax.lax.dot_general(
                h, w, (((1,), (1,)), ((), ()))) + b
            ref[2 * hh] = qq[:, :DH]
            ref[2 * hh + 1] = qq[:, DH:]


def _k1b_body(q_ref, k_ref, v_ref, ao_ref):
    s = jax.lax.dot_general(q_ref[0], k_ref[0],
                            (((1,), (1,)), ((), ()))) * SCALE
    p = jax.nn.softmax(s, axis=-1)
    ao_ref[0] = jnp.dot(p, v_ref[0])


def _k1c_body(h_ref, ao_ref, wo_ref, bo_ref, g1_ref, be1_ref, gatew_ref,
              h1_ref, meta_ref, perm_ref, beoff_ref):
    ao = bo_ref[...]
    for hh in range(H):
        ao = ao + jax.lax.dot_general(
            ao_ref[hh], wo_ref[:, hh * DH:(hh + 1) * DH],
            (((1,), (1,)), ((), ())))
    h1 = _ln(h_ref[...] + ao, g1_ref[...], be1_ref[...])
    h1_ref[...] = h1

    logits = jax.lax.dot_general(h1, gatew_ref[...], (((1,), (1,)), ((), ())))
    probs = jax.nn.softmax(logits, axis=-1)
    lane = jax.lax.broadcasted_iota(jnp.int32, (S, E), 1)
    m1 = jnp.max(probs, axis=-1, keepdims=True)
    i1 = jnp.min(jnp.where(probs == m1, lane, E), axis=-1, keepdims=True)
    probs2 = jnp.where(lane == i1, -1.0, probs)
    m2 = jnp.max(probs2, axis=-1, keepdims=True)
    i2 = jnp.min(jnp.where(probs2 == m2, lane, E), axis=-1, keepdims=True)
    denom = m1 + m2 + 1e-8
    wA = m1 / denom
    wB = m2 / denom

    a = jnp.where((lane == i1) | (lane == i2), 1.0, 0.0)
    ltri = jnp.where(
        jax.lax.broadcasted_iota(jnp.int32, (S, S), 0)
        > jax.lax.broadcasted_iota(jnp.int32, (S, S), 1), 1.0, 0.0)
    rank = jax.lax.dot_general(ltri, a,
                               (((1,), (0,)), ((), ())))
    counts = jnp.sum(a, axis=0, keepdims=True)
    cpad = jnp.ceil(counts / BLK) * BLK
    offs = jnp.zeros_like(cpad)
    for e in range(1, E):
        offs = offs + jnp.where(lane[:1] >= e, cpad[:, e - 1:e], 0.0)
    oh1 = jnp.where(lane == i1, 1.0, 0.0)
    oh2 = jnp.where(lane == i2, 1.0, 0.0)
    d1 = jnp.sum(oh1 * (rank + offs), axis=-1, keepdims=True)
    d2 = jnp.sum(oh2 * (rank + offs), axis=-1, keepdims=True)
    lane_p = jax.lax.broadcasted_iota(jnp.int32, (S, 128), 1)
    meta = jnp.where(lane_p == 0, d1, 0.0)
    meta = jnp.where(lane_p == 1, d2, meta)
    meta = jnp.where(lane_p == 2, wA, meta)
    meta = jnp.where(lane_p == 3, wB, meta)
    meta_ref[...] = meta

    tids = jax.lax.broadcasted_iota(jnp.int32, (1, S), 1)
    thi = (tids // 128).astype(jnp.float32)
    tlo = (tids % 128).astype(jnp.float32)
    ones_r = jnp.full((1, S), 1.0, jnp.float32)
    lane_b = jax.lax.broadcasted_iota(jnp.int32, (S, BLK), 1).astype(jnp.float32)
    lane_r = jax.lax.broadcasted_iota(jnp.int32, (1, BLK), 1)
    dd = (((1,), (0,)), ((), ()))
    for pb in range(NB):
        pv = lane_b + (pb * BLK)
        ind = jnp.where((d1 == pv) | (d2 == pv), 1.0, 0.0)
        pr = (jax.lax.dot_general(thi, ind, dd) * 128.0
              + jax.lax.dot_general(tlo, ind, dd))
        hit = jax.lax.dot_general(ones_r, ind, dd)
        dummy = ((lane_r + pb * BLK) % S).astype(jnp.float32)
        pr = pr + (1.0 - hit) * dummy
        perm_ref[pb:pb + 1, :] = pr.astype(jnp.int32)

    coli = jax.lax.broadcasted_iota(jnp.int32, (8, 128), 1).astype(jnp.float32) * BLK
    be = jnp.zeros((8, 128), jnp.float32)
    offs_b = offs + cpad
    for e in range(1, E):
        be = be + jnp.where(coli >= offs_b[:1, e - 1:e], 1.0, 0.0)
    nbu = jnp.sum(cpad, axis=-1, keepdims=True) / BLK
    beoff_ref[...] = jnp.where(
        jax.lax.broadcasted_iota(jnp.int32, (8, 128), 0) == 0,
        be, nbu).astype(jnp.int32)


NWORK = 32
GCHUNK = 64


def _sc_gather(table, idx):
    M = idx.shape[0]
    per_w = M // NWORK
    nch = per_w // GCHUNK
    mesh = plsc.VectorSubcoreMesh(core_axis_name="c", subcore_axis_name="s")

    @functools.partial(
        pl.kernel, mesh=mesh,
        out_type=jax.ShapeDtypeStruct((M, D), jnp.float32),
        scratch_types=[
            pltpu.VMEM((per_w,), jnp.int32),
            pltpu.VMEM((GCHUNK, D), jnp.float32),
            pltpu.VMEM((GCHUNK, D), jnp.float32),
            pltpu.SemaphoreType.DMA,
            pltpu.SemaphoreType.DMA,
        ],
    )
    def k(table_hbm, idx_hbm, out_hbm, idx_v, rows0, rows1, sem0, sem1):
        wid = lax.axis_index("s") * 2 + lax.axis_index("c")
        base = wid * per_w
        pltpu.sync_copy(idx_hbm.at[pl.ds(base, per_w)], idx_v)
        bufs = (rows0, rows1)
        sems = (sem0, sem1)
        cps = []
        for c in range(nch):
            cps.append(pltpu.async_copy(
                table_hbm.at[idx_v.at[pl.ds(c * GCHUNK, GCHUNK)]],
                bufs[c % 2], sems[c % 2]))
            if c >= 1:
                cps[c - 1].wait()
                pltpu.sync_copy(bufs[(c - 1) % 2],
                                out_hbm.at[pl.ds(base + (c - 1) * GCHUNK,
                                                 GCHUNK)])
        cps[nch - 1].wait()
        pltpu.sync_copy(bufs[(nch - 1) % 2],
                        out_hbm.at[pl.ds(base + (nch - 1) * GCHUNK, GCHUNK)])

    return k(table, idx)


def _sc_gather_pair(table, idx1, idx2):
    M = idx1.shape[0]
    per_w = M // NWORK
    mesh = plsc.VectorSubcoreMesh(core_axis_name="c", subcore_axis_name="s")

    @functools.partial(
        pl.kernel, mesh=mesh,
        out_type=[jax.ShapeDtypeStruct((M, D), jnp.float32),
                  jax.ShapeDtypeStruct((M, D), jnp.float32)],
        scratch_types=[
            pltpu.VMEM((per_w,), jnp.int32),
            pltpu.VMEM((per_w,), jnp.int32),
            pltpu.VMEM((per_w, D), jnp.float32),
            pltpu.VMEM((per_w, D), jnp.float32),
            pltpu.SemaphoreType.DMA,
            pltpu.SemaphoreType.DMA,
        ],
    )
    def k(table_hbm, i1_hbm, i2_hbm, o1_hbm, o2_hbm,
          i1_v, i2_v, r1_v, r2_v, sem1, sem2):
        wid = lax.axis_index("s") * 2 + lax.axis_index("c")
        base = wid * per_w
        pltpu.sync_copy(i1_hbm.at[pl.ds(base, per_w)], i1_v)
        pltpu.sync_copy(i2_hbm.at[pl.ds(base, per_w)], i2_v)
        cp1 = pltpu.async_copy(table_hbm.at[i1_v], r1_v, sem1)
        cp2 = pltpu.async_copy(table_hbm.at[i2_v], r2_v, sem2)
        cp1.wait()
        pltpu.sync_copy(r1_v, o1_hbm.at[pl.ds(base, per_w)])
        cp2.wait()
        pltpu.sync_copy(r2_v, o2_hbm.at[pl.ds(base, per_w)])

    return k(table, idx1, idx2)


def _sc_gather3(table, idx3):
    M = idx3.shape[1]
    per_w = M // NWORK
    mesh = plsc.VectorSubcoreMesh(core_axis_name="c", subcore_axis_name="s")
    out_sd = jax.ShapeDtypeStruct((M, D), jnp.float32)

    @functools.partial(
        pl.kernel, mesh=mesh,
        out_type=[out_sd, out_sd, out_sd],
        scratch_types=[
            pltpu.VMEM((per_w,), jnp.int32),
            pltpu.VMEM((per_w,), jnp.int32),
            pltpu.VMEM((per_w, D), jnp.float32),
            pltpu.VMEM((per_w, D), jnp.float32),
            pltpu.SemaphoreType.DMA,
            pltpu.SemaphoreType.DMA,
        ],
    )
    def k(table_hbm, idx_hbm, o0_hbm, o1_hbm, o2_hbm,
          ia_v, ib_v, ra_v, rb_v, sema, semb):
        wid = lax.axis_index("s") * 2 + lax.axis_index("c")
        base = wid * per_w
        outs = (o0_hbm, o1_hbm, o2_hbm)
        ibufs = (ia_v, ib_v)
        rbufs = (ra_v, rb_v)
        sems = (sema, semb)
        cps = [None, None, None]
        for j in range(3):
            pltpu.sync_copy(idx_hbm.at[j, pl.ds(base, per_w)], ibufs[j % 2])
            cps[j] = pltpu.async_copy(table_hbm.at[ibufs[j % 2]],
                                      rbufs[j % 2], sems[j % 2])
            if j >= 1:
                cps[j - 1].wait()
                pltpu.sync_copy(rbufs[(j - 1) % 2],
                                outs[j - 1].at[pl.ds(base, per_w)])
        cps[2].wait()
        pltpu.sync_copy(rbufs[0], outs[2].at[pl.ds(base, per_w)])

    return k(table, idx3)


def _k2_body(be_ref, xg_ref, w1_ref, b1_ref, w2_ref, b2_ref, out_ref):
    i = pl.program_id(0)

    @pl.when(i < be_ref[NB])
    def _():
        x = xg_ref[...]
        hid = jax.lax.dot_general(x, w1_ref[0], (((1,), (1,)), ((), ())))
        hid = _gelu(hid + b1_ref[0])
        ye = jax.lax.dot_general(hid, w2_ref[0], (((1,), (1,)), ((), ())))
        out_ref[...] = ye + b2_ref[0]


def _k3_body(h1_ref, r1_ref, r2_ref, meta_ref, g2_ref, be2_ref, gc_ref,
             bc_ref, wc1_ref, bc1_ref, wc2_ref, bc2_ref, out_ref):
    wa = meta_ref[:, 2:3]
    wb = meta_ref[:, 3:4]
    moe = wa * r1_ref[...] + wb * r2_ref[...]
    h2 = _ln(h1_ref[...] + moe, g2_ref[...], be2_ref[...])
    pooled = jnp.mean(h2, axis=0, keepdims=True)
    c = _ln(pooled, gc_ref[...], bc_ref[...])
    c = jax.lax.dot_general(c, wc1_ref[...], (((1,), (1,)), ((), ())))
    c = _gelu(c + bc1_ref[...])
    out_ref[...] = jnp.sum(c * wc2_ref[...], axis=-1, keepdims=True) + bc2_ref[...]


def _pe_table():
    position = jnp.arange(S, dtype=jnp.float32)[:, None]
    div_term = jnp.exp(jnp.arange(0, D, 2, dtype=jnp.float32)
                       * (-math.log(10000.0) / D))
    pe = jnp.zeros((S, D), dtype=jnp.float32)
    pe = pe.at[:, 0::2].set(jnp.sin(position * div_term))
    pe = pe.at[:, 1::2].set(jnp.cos(position * div_term))
    return pe


def kernel(x, Wproj, bproj, Wqkv, bqkv, Wo, bo, g1, be1, gateW, W1, b1,
           W2, b2, g2, be2, gc, bc, Wc1, bc1, Wc2, bc2):
    xr = x.reshape(S, IN)
    pe = _pe_table()
    f32 = jnp.float32
    h, q3, k3, v3 = pl.pallas_call(
        _k1a_body,
        out_shape=[jax.ShapeDtypeStruct((S, D), f32),
                   jax.ShapeDtypeStruct((H, S, DH), f32),
                   jax.ShapeDtypeStruct((H, S, DH), f32),
                   jax.ShapeDtypeStruct((H, S, DH), f32)],
    )(xr, pe, Wproj, bproj.reshape(1, D), Wqkv, bqkv.reshape(1, 3 * D))

    ao = pl.pallas_call(
        _k1b_body,
        grid=(H,),
        in_specs=[
            pl.BlockSpec((1, S, DH), lambda hh: (hh, 0, 0)),
            pl.BlockSpec((1, S, DH), lambda hh: (hh, 0, 0)),
            pl.BlockSpec((1, S, DH), lambda hh: (hh, 0, 0)),
        ],
        out_specs=pl.BlockSpec((1, S, DH), lambda hh: (hh, 0, 0)),
        out_shape=jax.ShapeDtypeStruct((H, S, DH), f32),
        compiler_params=pltpu.CompilerParams(
            dimension_semantics=("parallel",)),
    )(q3, k3, v3)

    h1, meta, perm, beoff = pl.pallas_call(
        _k1c_body,
        out_shape=[jax.ShapeDtypeStruct((S, D), f32),
                   jax.ShapeDtypeStruct((S, 128), f32),
                   jax.ShapeDtypeStruct((NB, BLK), jnp.int32),
                   jax.ShapeDtypeStruct((8, 128), jnp.int32)],
    )(h, ao, Wo, bo.reshape(1, D), g1.reshape(1, D), be1.reshape(1, D),
      gateW)

    be_nbu = jnp.concatenate(
        [beoff[0, :NB], beoff[1, :1]]).astype(jnp.int32)
    perm_flat = perm.reshape(P)
    d1 = meta[:, 0].astype(jnp.int32)
    d2 = meta[:, 1].astype(jnp.int32)

    xg_parts = _sc_gather3(h1, perm.reshape(3, P // 3))
    xg = jnp.concatenate(xg_parts, axis=0)

    moe_s = pl.pallas_call(
        _k2_body,
        grid_spec=pltpu.PrefetchScalarGridSpec(
            num_scalar_prefetch=1,
            grid=(NB,),
            in_specs=[
                pl.BlockSpec((BLK, D), lambda i, be: (i, 0)),
                pl.BlockSpec((1, DFF, D), lambda i, be: (be[i], 0, 0)),
                pl.BlockSpec((1, 1, DFF), lambda i, be: (be[i], 0, 0)),
                pl.BlockSpec((1, D, DFF), lambda i, be: (be[i], 0, 0)),
                pl.BlockSpec((1, 1, D), lambda i, be: (be[i], 0, 0)),
            ],
            out_specs=pl.BlockSpec((BLK, D), lambda i, be: (i, 0)),
        ),
        out_shape=jax.ShapeDtypeStruct((P, D), f32),
        compiler_params=pltpu.CompilerParams(
            dimension_semantics=("arbitrary",)),
    )(be_nbu, xg, W1, b1.reshape(E, 1, DFF), W2, b2.reshape(E, 1, D))

    r1, r2 = _sc_gather_pair(moe_s, d1, d2)

    out = pl.pallas_call(
        _k3_body,
        out_shape=jax.ShapeDtypeStruct((1, 1), f32),
    )(h1, r1, r2, meta, g2.reshape(1, D), be2.reshape(1, D),
      gc.reshape(1, D), bc.reshape(1, D), Wc1, bc1.reshape(1, D // 2),
      Wc2, bc2.reshape(1, 1))
    return out.reshape(B)
